# baseline (speedup 1.0000x reference)
import functools
import jax
import jax.numpy as jnp
from jax import lax
from jax.experimental import pallas as pl
from jax.experimental.pallas import tpu as pltpu
from jax.experimental.pallas import tpu_sc as plsc

N = 50000
D = 128
NS = 6
E_PS = 50000
E_LR = 10000

NC = 2
NSUB = 16
NW = NC * NSUB

CH = 4096
NCH = 14
NPAD = CH * NCH

E_TOT = 2 * NS * E_PS + 2 * E_LR
EW = 19456
E_PAD = EW * NW
BB = 128
SBMAX = 20
NBMAX = SBMAX * 8
CAPW = NBMAX * 2 * BB

BLK = 2048
NBLK = NPAD // BLK

def _mesh():
  return plsc.VectorSubcoreMesh(core_axis_name="c", subcore_axis_name="s")


def _bin_body(u_hbm, v_hbm, buv_hbm, cnt_hbm, u_v, v_v, ouv_v, cnt_v):
  cid = lax.axis_index("c")
  sid = lax.axis_index("s")
  lane = lax.iota(jnp.int32, 16)
  w = sid * NC + cid
  pltpu.sync_copy(u_hbm.at[pl.ds(w * EW, EW)], u_v)
  pltpu.sync_copy(v_hbm.at[pl.ds(w * EW, EW)], v_v)
  counts = jnp.zeros((16,), jnp.int32)
  trash_u = jnp.full((16,), CH, jnp.int32)
  zero_v = jnp.zeros((16,), jnp.int32)
  for c in range(NCH):
    base = c * CH

    def body(i, off):
      u16 = u_v[pl.ds(i * 16, 16)]
      v16 = v_v[pl.ds(i * 16, 16)]
      m = (u16 >= base) & (u16 < base + CH)
      pc = plsc.cumsum(m.astype(jnp.int32))
      e = jnp.where(m, off + pc - 1, off)
      zero = jnp.zeros((16,), jnp.int32)
      plsc.store_scatter(ouv_v, [e // BB, zero, e % BB], u16 - base, mask=m)
      plsc.store_scatter(ouv_v, [e // BB, zero + 1, e % BB], v16, mask=m)
      return off + jnp.sum(m.astype(jnp.int32))

    off = lax.fori_loop(0, EW // 16, body, jnp.int32(0))
    for k in range(BB // 16):
      e = off + k * 16 + lane
      zero = jnp.zeros((16,), jnp.int32)
      plsc.store_scatter(ouv_v, [e // BB, zero, e % BB], trash_u)
      plsc.store_scatter(ouv_v, [e // BB, zero + 1, e % BB], zero_v)
    cnt = ((off + BB - 1) // BB) * BB
    counts = jnp.where(lane == c, cnt, counts)
    pltpu.sync_copy(ouv_v, buv_hbm.at[w, c])
  cnt_v[...] = counts
  pltpu.sync_copy(cnt_v, cnt_hbm.at[w])


def _bin_kernel(u_all, v_all):
  f = pl.kernel(
      _bin_body,
      out_type=(
          jax.ShapeDtypeStruct((NW, NCH, NBMAX, 2, BB), jnp.int32),
          jax.ShapeDtypeStruct((NW, 16), jnp.int32),
      ),
      mesh=_mesh(),
      scratch_types=[
          pltpu.VMEM((EW,), jnp.int32),
          pltpu.VMEM((EW,), jnp.int32),
          pltpu.VMEM((NBMAX, 2, BB), jnp.int32),
          pltpu.VMEM((16,), jnp.int32),
      ],
      compiler_params=pltpu.CompilerParams(needs_layout_passes=False),
  )
  return f(u_all, v_all)


def _combine_body(yctr_hbm, yed_hbm, buv_hbm, cnt_hbm, temp_hbm,
                  acc, ixb, rows, cnt_v, gsem, ssem):
  cid = lax.axis_index("c")
  sid = lax.axis_index("s")
  lane = lax.iota(jnp.int32, 16)
  rps = CH // NSUB
  for c in range(NCH):
    @pl.when((c % NC) == cid)
    def _():
      base = c * CH
      pltpu.sync_copy(
          yctr_hbm.at[pl.ds(base + sid * rps, rps)],
          acc.at[pl.ds(sid * rps, rps)])
      plsc.subcore_barrier()
      for wj in range(NW // NSUB):
        w = sid * (NW // NSUB) + wj
        pltpu.sync_copy(cnt_hbm.at[w], cnt_v)
        kcnt = jnp.max(jnp.where(lane == c, cnt_v[...], 0))
        nb = kcnt // BB

        def body(sb, carry):
          pltpu.sync_copy(buv_hbm.at[w, c, pl.ds(sb * 8, 8)], ixb)
          for h in range(2):
            for k in range(4):
              j = sb * 8 + h * 4 + k
              @pl.when(j < nb)
              def _():
                q = h * 4 + k
                pltpu.async_copy(
                    yed_hbm.at[ixb.at[q, 1, pl.ds(0, BB // 2)]],
                    rows.at[k].at[pl.ds(0, BB // 2)], gsem.at[2 * k])
                pltpu.async_copy(
                    yed_hbm.at[ixb.at[q, 1, pl.ds(BB // 2, BB // 2)]],
                    rows.at[k].at[pl.ds(BB // 2, BB // 2)], gsem.at[2 * k + 1])
            for k in range(4):
              j = sb * 8 + h * 4 + k
              @pl.when(j < nb)
              def _():
                q = h * 4 + k
                pltpu.make_async_copy(
                    yed_hbm.at[ixb.at[q, 1, pl.ds(0, BB // 2)]],
                    rows.at[k].at[pl.ds(0, BB // 2)], gsem.at[2 * k]).wait()
                pltpu.make_async_copy(
                    yed_hbm.at[ixb.at[q, 1, pl.ds(BB // 2, BB // 2)]],
                    rows.at[k].at[pl.ds(BB // 2, BB // 2)],
                    gsem.at[2 * k + 1]).wait()
                pltpu.async_copy(
                    rows.at[k], acc.at[ixb.at[q, 0]],
                    ssem.at[k], add=True)
            for k in range(4):
              j = sb * 8 + h * 4 + k
              @pl.when(j < nb)
              def _():
                q = h * 4 + k
                pltpu.make_async_copy(
                    rows.at[k], acc.at[ixb.at[q, 0]],
                    ssem.at[k]).wait()
          return carry

        lax.fori_loop(0, (nb + 7) // 8, body, jnp.int32(0))
      plsc.subcore_barrier()
      pltpu.sync_copy(
          acc.at[pl.ds(sid * rps, rps)],
          temp_hbm.at[pl.ds(base + sid * rps, rps)])


def _combine(yctr, yed, buv, cnt):
  f = pl.kernel(
      _combine_body,
      out_type=jax.ShapeDtypeStruct((NPAD, D), jnp.float32),
      mesh=_mesh(),
      scratch_types=[
          pltpu.VMEM_SHARED((CH + 8, D), jnp.float32),
          pltpu.VMEM((8, 2, BB), jnp.int32),
          pltpu.VMEM((4, BB, D), jnp.float32),
          pltpu.VMEM((16,), jnp.int32),
          pltpu.SemaphoreType.DMA((8,)),
          pltpu.SemaphoreType.DMA((4,)),
      ],
      compiler_params=pltpu.CompilerParams(needs_layout_passes=False),
  )
  return f(yctr, yed, buv, cnt)


def _in_body(c_ref, f_ref, iw1, ib1, iw2, ib2, sw1, sb1, sw2, sb2, o_ref):
  c = c_ref[...]
  f = f_ref[...]
  h1 = jnp.maximum(c[:, 0:1] * iw1[0:1, :] + c[:, 1:2] * iw1[1:2, :]
                   + ib1[...], 0.0)
  y1 = jnp.dot(h1, iw2[...], preferred_element_type=jnp.float32) + ib2[...]
  h2 = jnp.maximum(f[:, 0:1] * sw1[0:1, :] + f[:, 1:2] * sw1[1:2, :]
                   + sb1[...], 0.0)
  y2 = jnp.dot(h2, sw2[...], preferred_element_type=jnp.float32) + sb2[...]
  o_ref[...] = jnp.maximum(y1 + y2, 0.0)


def _input_mlp(ctrs_p, feats_p, iw1, ib1, iw2, ib2, sw1, sb1, sw2, sb2):
  full = lambda s: pl.BlockSpec(s, lambda j: tuple(0 for _ in s))
  return pl.pallas_call(
      _in_body,
      grid=(NBLK,),
      in_specs=[
          pl.BlockSpec((BLK, 2), lambda j: (j, 0)),
          pl.BlockSpec((BLK, 2), lambda j: (j, 0)),
          full((2, D)), full((1, D)), full((D, D)), full((1, D)),
          full((2, D)), full((1, D)), full((D, D)), full((1, D)),
      ],
      out_specs=pl.BlockSpec((BLK, D), lambda j: (j, 0)),
      out_shape=jax.ShapeDtypeStruct((NPAD, D), jnp.float32),
  )(ctrs_p, feats_p, iw1, ib1, iw2, ib2, sw1, sb1, sw2, sb2)


def _yc_body(f_ref, w_ref, o_ref):
  o_ref[...] = jnp.dot(f_ref[...], w_ref[...], preferred_element_type=jnp.float32)


def _y_ctr(feat, w):
  full = lambda s: pl.BlockSpec(s, lambda j: tuple(0 for _ in s))
  return pl.pallas_call(
      _yc_body,
      grid=(NBLK,),
      in_specs=[pl.BlockSpec((BLK, D), lambda j: (j, 0)), full((D, D))],
      out_specs=pl.BlockSpec((BLK, D), lambda j: (j, 0)),
      out_shape=jax.ShapeDtypeStruct((NPAD, D), jnp.float32),
  )(feat, w)


def _ye_body(f_ref, w_ref, o_ref):
  o_ref[...] = jnp.dot(f_ref[...], w_ref[0], preferred_element_type=jnp.float32)


def _y_edges(feat, wed):
  return pl.pallas_call(
      _ye_body,
      grid=(NBLK, 14),
      in_specs=[
          pl.BlockSpec((BLK, D), lambda j, k: (j, 0)),
          pl.BlockSpec((1, D, D), lambda j, k: (k, 0, 0)),
      ],
      out_specs=pl.BlockSpec((BLK, D), lambda j, k: (k * NBLK + j, 0)),
      out_shape=jax.ShapeDtypeStruct((14 * NPAD, D), jnp.float32),
  )(feat, wed)


def _post_body(t_ref, r_ref, g_ref, b_ref, w1, b1, w2, b2, o_ref):
  t = t_ref[...]
  mu = jnp.mean(t, axis=1, keepdims=True)
  var = jnp.mean((t - mu) * (t - mu), axis=1, keepdims=True)
  xn = (t - mu) * lax.rsqrt(var + 1e-5) * g_ref[...] + b_ref[...]
  h = jnp.maximum(xn, 0.0)
  h = jnp.maximum(jnp.dot(h, w1[...], preferred_element_type=jnp.float32)
                  + b1[...], 0.0)
  y = jnp.dot(h, w2[...], preferred_element_type=jnp.float32) + b2[...]
  o_ref[...] = jnp.maximum(y + r_ref[...], 0.0)


def _post(temp, res, g, b, w1, b1, w2, b2):
  full = lambda s: pl.BlockSpec(s, lambda j: tuple(0 for _ in s))
  return pl.pallas_call(
      _post_body,
      grid=(NBLK,),
      in_specs=[
          pl.BlockSpec((BLK, D), lambda j: (j, 0)),
          pl.BlockSpec((BLK, D), lambda j: (j, 0)),
          full((1, D)), full((1, D)), full((D, D)), full((1, D)),
          full((D, D)), full((1, D)),
      ],
      out_specs=pl.BlockSpec((BLK, D), lambda j: (j, 0)),
      out_shape=jax.ShapeDtypeStruct((NPAD, D), jnp.float32),
  )(temp, res, g, b, w1, b1, w2, b2)


def kernel(feats, ctrs, pre_u, pre_v, suc_u, suc_v, left_u, left_v, right_u,
           right_v, idcs, in_W1, in_b1, in_W2, in_b2, seg_W1, seg_b1, seg_W2,
           seg_b2, ctr_W, pre_W, suc_W, left_W, right_W, gn_g, gn_b, ctr2_W1,
           ctr2_b1, ctr2_W2, ctr2_b2):
  i32 = jnp.int32
  pad = lambda x: jnp.pad(x, ((0, NPAD - N), (0, 0)))
  ctrs_p = pad(ctrs)
  feats_p = pad(feats)

  u_all = jnp.concatenate([
      pre_u.reshape(-1), suc_u.reshape(-1), left_u, right_u]).astype(i32)
  pre_off = jnp.arange(NS, dtype=i32)[:, None] * NPAD
  suc_off = (NS + jnp.arange(NS, dtype=i32))[:, None] * NPAD
  v_all = jnp.concatenate([
      (pre_v.astype(i32) + pre_off).reshape(-1),
      (suc_v.astype(i32) + suc_off).reshape(-1),
      left_v.astype(i32) + 2 * NS * NPAD,
      right_v.astype(i32) + (2 * NS + 1) * NPAD,
  ])
  u_all = jnp.pad(u_all, (0, E_PAD - E_TOT), constant_values=2**30)
  v_all = jnp.pad(v_all, (0, E_PAD - E_TOT))

  wed = jnp.concatenate([
      pre_W, suc_W, left_W[:, None], right_W[:, None]], axis=1)

  r2 = lambda x: x.reshape(1, -1)

  buv, cnt = _bin_kernel(u_all, v_all)

  feat = _input_mlp(ctrs_p, feats_p, in_W1, r2(in_b1), in_W2, r2(in_b2),
                    seg_W1, r2(seg_b1), seg_W2, r2(seg_b2))
  res = feat
  for i in range(4):
    yctr = _y_ctr(feat, ctr_W[i])
    yed = _y_edges(feat, wed[i])
    temp = _combine(yctr, yed, buv, cnt)
    feat = _post(temp, res, r2(gn_g[i]), r2(gn_b[i]), ctr2_W1[i],
                 r2(ctr2_b1[i]), ctr2_W2[i], r2(ctr2_b2[i]))
    res = feat
  return feat[:N]


# speedup vs baseline: 1.0200x; 1.0200x over previous
import functools
import jax
import jax.numpy as jnp
from jax import lax
from jax.experimental import pallas as pl
from jax.experimental.pallas import tpu as pltpu
from jax.experimental.pallas import tpu_sc as plsc

N = 50000
D = 128
NS = 6
E_PS = 50000
E_LR = 10000

NC = 2
NSUB = 16
NW = NC * NSUB

CH = 3584
NCH = 14
NPAD = CH * NCH

E_TOT = 2 * NS * E_PS + 2 * E_LR
EW = 19456
E_PAD = EW * NW
BB = 128
SBMAX = 20
NBMAX = SBMAX * 8
CAPW = NBMAX * 2 * BB

BLK = 1792
NBLK = NPAD // BLK

def _mesh():
  return plsc.VectorSubcoreMesh(core_axis_name="c", subcore_axis_name="s")


def _bin_body(u_hbm, v_hbm, buv_hbm, cnt_hbm, u_v, v_v, ouv_v, cnt_v):
  cid = lax.axis_index("c")
  sid = lax.axis_index("s")
  lane = lax.iota(jnp.int32, 16)
  w = sid * NC + cid
  pltpu.sync_copy(u_hbm.at[pl.ds(w * EW, EW)], u_v)
  pltpu.sync_copy(v_hbm.at[pl.ds(w * EW, EW)], v_v)
  counts = jnp.zeros((16,), jnp.int32)
  trash_u = jnp.full((16,), CH, jnp.int32)
  zero_v = jnp.zeros((16,), jnp.int32)
  for c in range(NCH):
    base = c * CH

    def body(i, off):
      u16 = u_v[pl.ds(i * 16, 16)]
      v16 = v_v[pl.ds(i * 16, 16)]
      m = (u16 >= base) & (u16 < base + CH)
      pc = plsc.cumsum(m.astype(jnp.int32))
      e = jnp.where(m, off + pc - 1, off)
      zero = jnp.zeros((16,), jnp.int32)
      plsc.store_scatter(ouv_v, [e // BB, zero, e % BB], u16 - base, mask=m)
      plsc.store_scatter(ouv_v, [e // BB, zero + 1, e % BB], v16, mask=m)
      return off + jnp.sum(m.astype(jnp.int32))

    off = lax.fori_loop(0, EW // 16, body, jnp.int32(0))
    for k in range(BB // 16):
      e = off + k * 16 + lane
      zero = jnp.zeros((16,), jnp.int32)
      plsc.store_scatter(ouv_v, [e // BB, zero, e % BB], trash_u)
      plsc.store_scatter(ouv_v, [e // BB, zero + 1, e % BB], zero_v)
    cnt = ((off + BB - 1) // BB) * BB
    counts = jnp.where(lane == c, cnt, counts)
    pltpu.sync_copy(ouv_v, buv_hbm.at[w, c])
  cnt_v[...] = counts
  pltpu.sync_copy(cnt_v, cnt_hbm.at[w])


def _bin_kernel(u_all, v_all):
  f = pl.kernel(
      _bin_body,
      out_type=(
          jax.ShapeDtypeStruct((NW, NCH, NBMAX, 2, BB), jnp.int32),
          jax.ShapeDtypeStruct((NW, 16), jnp.int32),
      ),
      mesh=_mesh(),
      scratch_types=[
          pltpu.VMEM((EW,), jnp.int32),
          pltpu.VMEM((EW,), jnp.int32),
          pltpu.VMEM((NBMAX, 2, BB), jnp.int32),
          pltpu.VMEM((16,), jnp.int32),
      ],
      compiler_params=pltpu.CompilerParams(needs_layout_passes=False),
  )
  return f(u_all, v_all)


def _combine_body(yctr_hbm, yed_hbm, buv_hbm, cnt_hbm, temp_hbm,
                  acc, ixb, rows, cnt_v, gsem, ssem):
  cid = lax.axis_index("c")
  sid = lax.axis_index("s")
  lane = lax.iota(jnp.int32, 16)
  rps = CH // NSUB
  for c in range(NCH):
    @pl.when((c % NC) == cid)
    def _():
      base = c * CH
      pltpu.sync_copy(
          yctr_hbm.at[pl.ds(base + sid * rps, rps)],
          acc.at[pl.ds(sid * rps, rps)])
      plsc.subcore_barrier()
      for wj in range(NW // NSUB):
        w = sid * (NW // NSUB) + wj
        pltpu.sync_copy(cnt_hbm.at[w], cnt_v)
        kcnt = jnp.max(jnp.where(lane == c, cnt_v[...], 0))
        nb = kcnt // BB

        def body(sb, carry):
          pltpu.sync_copy(buv_hbm.at[w, c, pl.ds(sb * 8, 8)], ixb)
          for h in range(2):
            for k in range(4):
              j = sb * 8 + h * 4 + k
              @pl.when(j < nb)
              def _():
                q = h * 4 + k
                pltpu.async_copy(
                    yed_hbm.at[ixb.at[q, 1]], rows.at[k], gsem.at[k])
            for k in range(4):
              j = sb * 8 + h * 4 + k
              @pl.when(j < nb)
              def _():
                q = h * 4 + k
                pltpu.make_async_copy(
                    yed_hbm.at[ixb.at[q, 1]], rows.at[k], gsem.at[k]).wait()
                pltpu.async_copy(
                    rows.at[k], acc.at[ixb.at[q, 0]],
                    ssem.at[k], add=True)
            for k in range(4):
              j = sb * 8 + h * 4 + k
              @pl.when(j < nb)
              def _():
                q = h * 4 + k
                pltpu.make_async_copy(
                    rows.at[k], acc.at[ixb.at[q, 0]],
                    ssem.at[k]).wait()
          return carry

        lax.fori_loop(0, (nb + 7) // 8, body, jnp.int32(0))
      plsc.subcore_barrier()
      pltpu.sync_copy(
          acc.at[pl.ds(sid * rps, rps)],
          temp_hbm.at[pl.ds(base + sid * rps, rps)])


def _combine(yctr, yed, buv, cnt):
  f = pl.kernel(
      _combine_body,
      out_type=jax.ShapeDtypeStruct((NPAD, D), jnp.float32),
      mesh=_mesh(),
      scratch_types=[
          pltpu.VMEM_SHARED((CH + 8, D), jnp.float32),
          pltpu.VMEM((8, 2, BB), jnp.int32),
          pltpu.VMEM((4, BB, D), jnp.float32),
          pltpu.VMEM((16,), jnp.int32),
          pltpu.SemaphoreType.DMA((4,)),
          pltpu.SemaphoreType.DMA((4,)),
      ],
      compiler_params=pltpu.CompilerParams(needs_layout_passes=False),
  )
  return f(yctr, yed, buv, cnt)


def _in_body(c_ref, f_ref, iw1, ib1, iw2, ib2, sw1, sb1, sw2, sb2, o_ref):
  c = c_ref[...]
  f = f_ref[...]
  h1 = jnp.maximum(c[:, 0:1] * iw1[0:1, :] + c[:, 1:2] * iw1[1:2, :]
                   + ib1[...], 0.0)
  y1 = jnp.dot(h1, iw2[...], preferred_element_type=jnp.float32) + ib2[...]
  h2 = jnp.maximum(f[:, 0:1] * sw1[0:1, :] + f[:, 1:2] * sw1[1:2, :]
                   + sb1[...], 0.0)
  y2 = jnp.dot(h2, sw2[...], preferred_element_type=jnp.float32) + sb2[...]
  o_ref[...] = jnp.maximum(y1 + y2, 0.0)


def _input_mlp(ctrs_p, feats_p, iw1, ib1, iw2, ib2, sw1, sb1, sw2, sb2):
  full = lambda s: pl.BlockSpec(s, lambda j: tuple(0 for _ in s))
  return pl.pallas_call(
      _in_body,
      grid=(NBLK,),
      in_specs=[
          pl.BlockSpec((BLK, 2), lambda j: (j, 0)),
          pl.BlockSpec((BLK, 2), lambda j: (j, 0)),
          full((2, D)), full((1, D)), full((D, D)), full((1, D)),
          full((2, D)), full((1, D)), full((D, D)), full((1, D)),
      ],
      out_specs=pl.BlockSpec((BLK, D), lambda j: (j, 0)),
      out_shape=jax.ShapeDtypeStruct((NPAD, D), jnp.float32),
  )(ctrs_p, feats_p, iw1, ib1, iw2, ib2, sw1, sb1, sw2, sb2)


def _yc_body(f_ref, w_ref, o_ref):
  o_ref[...] = jnp.dot(f_ref[...], w_ref[...], preferred_element_type=jnp.float32)


def _y_ctr(feat, w):
  full = lambda s: pl.BlockSpec(s, lambda j: tuple(0 for _ in s))
  return pl.pallas_call(
      _yc_body,
      grid=(NBLK,),
      in_specs=[pl.BlockSpec((BLK, D), lambda j: (j, 0)), full((D, D))],
      out_specs=pl.BlockSpec((BLK, D), lambda j: (j, 0)),
      out_shape=jax.ShapeDtypeStruct((NPAD, D), jnp.float32),
  )(feat, w)


def _ye_body(f_ref, w_ref, o_ref):
  o_ref[...] = jnp.dot(f_ref[...].astype(jnp.bfloat16),
                       w_ref[0].astype(jnp.bfloat16),
                       preferred_element_type=jnp.float32)


def _y_edges(feat, wed):
  return pl.pallas_call(
      _ye_body,
      grid=(NBLK, 14),
      in_specs=[
          pl.BlockSpec((BLK, D), lambda j, k: (j, 0)),
          pl.BlockSpec((1, D, D), lambda j, k: (k, 0, 0)),
      ],
      out_specs=pl.BlockSpec((BLK, D), lambda j, k: (k * NBLK + j, 0)),
      out_shape=jax.ShapeDtypeStruct((14 * NPAD, D), jnp.float32),
  )(feat, wed)


def _post_body(t_ref, r_ref, g_ref, b_ref, w1, b1, w2, b2, o_ref):
  t = t_ref[...]
  mu = jnp.mean(t, axis=1, keepdims=True)
  var = jnp.mean((t - mu) * (t - mu), axis=1, keepdims=True)
  xn = (t - mu) * lax.rsqrt(var + 1e-5) * g_ref[...] + b_ref[...]
  h = jnp.maximum(xn, 0.0)
  h = jnp.maximum(jnp.dot(h, w1[...], preferred_element_type=jnp.float32)
                  + b1[...], 0.0)
  y = jnp.dot(h, w2[...], preferred_element_type=jnp.float32) + b2[...]
  o_ref[...] = jnp.maximum(y + r_ref[...], 0.0)


def _post(temp, res, g, b, w1, b1, w2, b2):
  full = lambda s: pl.BlockSpec(s, lambda j: tuple(0 for _ in s))
  return pl.pallas_call(
      _post_body,
      grid=(NBLK,),
      in_specs=[
          pl.BlockSpec((BLK, D), lambda j: (j, 0)),
          pl.BlockSpec((BLK, D), lambda j: (j, 0)),
          full((1, D)), full((1, D)), full((D, D)), full((1, D)),
          full((D, D)), full((1, D)),
      ],
      out_specs=pl.BlockSpec((BLK, D), lambda j: (j, 0)),
      out_shape=jax.ShapeDtypeStruct((NPAD, D), jnp.float32),
  )(temp, res, g, b, w1, b1, w2, b2)


def kernel(feats, ctrs, pre_u, pre_v, suc_u, suc_v, left_u, left_v, right_u,
           right_v, idcs, in_W1, in_b1, in_W2, in_b2, seg_W1, seg_b1, seg_W2,
           seg_b2, ctr_W, pre_W, suc_W, left_W, right_W, gn_g, gn_b, ctr2_W1,
           ctr2_b1, ctr2_W2, ctr2_b2):
  i32 = jnp.int32
  pad = lambda x: jnp.pad(x, ((0, NPAD - N), (0, 0)))
  ctrs_p = pad(ctrs)
  feats_p = pad(feats)

  u_all = jnp.concatenate([
      pre_u.reshape(-1), suc_u.reshape(-1), left_u, right_u]).astype(i32)
  pre_off = jnp.arange(NS, dtype=i32)[:, None] * NPAD
  suc_off = (NS + jnp.arange(NS, dtype=i32))[:, None] * NPAD
  v_all = jnp.concatenate([
      (pre_v.astype(i32) + pre_off).reshape(-1),
      (suc_v.astype(i32) + suc_off).reshape(-1),
      left_v.astype(i32) + 2 * NS * NPAD,
      right_v.astype(i32) + (2 * NS + 1) * NPAD,
  ])
  u_all = jnp.pad(u_all, (0, E_PAD - E_TOT), constant_values=2**30)
  v_all = jnp.pad(v_all, (0, E_PAD - E_TOT))

  wed = jnp.concatenate([
      pre_W, suc_W, left_W[:, None], right_W[:, None]], axis=1)

  r2 = lambda x: x.reshape(1, -1)

  buv, cnt = _bin_kernel(u_all, v_all)

  feat = _input_mlp(ctrs_p, feats_p, in_W1, r2(in_b1), in_W2, r2(in_b2),
                    seg_W1, r2(seg_b1), seg_W2, r2(seg_b2))
  res = feat
  for i in range(4):
    yctr = _y_ctr(feat, ctr_W[i])
    yed = _y_edges(feat, wed[i])
    temp = _combine(yctr, yed, buv, cnt)
    feat = _post(temp, res, r2(gn_g[i]), r2(gn_b[i]), ctr2_W1[i],
                 r2(ctr2_b1[i]), ctr2_W2[i], r2(ctr2_b2[i]))
    res = feat
  return feat[:N]


# speedup vs baseline: 1.0217x; 1.0017x over previous
import functools
import jax
import jax.numpy as jnp
from jax import lax
from jax.experimental import pallas as pl
from jax.experimental.pallas import tpu as pltpu
from jax.experimental.pallas import tpu_sc as plsc

N = 50000
D = 128
NS = 6
E_PS = 50000
E_LR = 10000

NC = 2
NSUB = 16
NW = NC * NSUB

CH = 3584
NCH = 14
NPAD = CH * NCH

E_TOT = 2 * NS * E_PS + 2 * E_LR
EW = 19456
E_PAD = EW * NW
BB = 128
SBMAX = 20
NBMAX = SBMAX * 8
CAPW = NBMAX * 2 * BB

BLK = 1792
NBLK = NPAD // BLK

def _mesh():
  return plsc.VectorSubcoreMesh(core_axis_name="c", subcore_axis_name="s")


def _bin_body(u_hbm, v_hbm, buv_hbm, cnt_hbm, u_v, v_v, ouv_v, cnt_v):
  cid = lax.axis_index("c")
  sid = lax.axis_index("s")
  lane = lax.iota(jnp.int32, 16)
  w = sid * NC + cid
  pltpu.sync_copy(u_hbm.at[pl.ds(w * EW, EW)], u_v)
  pltpu.sync_copy(v_hbm.at[pl.ds(w * EW, EW)], v_v)
  counts = jnp.zeros((16,), jnp.int32)
  trash_u = jnp.full((16,), CH, jnp.int32)
  zero_v = jnp.zeros((16,), jnp.int32)
  for c in range(NCH):
    base = c * CH

    def body(i, off):
      u16 = u_v[pl.ds(i * 16, 16)]
      v16 = v_v[pl.ds(i * 16, 16)]
      m = (u16 >= base) & (u16 < base + CH)
      pc = plsc.cumsum(m.astype(jnp.int32))
      e = jnp.where(m, off + pc - 1, off)
      zero = jnp.zeros((16,), jnp.int32)
      plsc.store_scatter(ouv_v, [e // BB, zero, e % BB], u16 - base, mask=m)
      plsc.store_scatter(ouv_v, [e // BB, zero + 1, e % BB], v16, mask=m)
      return off + jnp.sum(m.astype(jnp.int32))

    off = lax.fori_loop(0, EW // 16, body, jnp.int32(0))
    for k in range(BB // 16):
      e = off + k * 16 + lane
      zero = jnp.zeros((16,), jnp.int32)
      plsc.store_scatter(ouv_v, [e // BB, zero, e % BB], trash_u)
      plsc.store_scatter(ouv_v, [e // BB, zero + 1, e % BB], zero_v)
    cnt = ((off + BB - 1) // BB) * BB
    counts = jnp.where(lane == c, cnt, counts)
    pltpu.sync_copy(ouv_v, buv_hbm.at[w, c])
  cnt_v[...] = counts
  pltpu.sync_copy(cnt_v, cnt_hbm.at[w])


def _bin_kernel(u_all, v_all):
  f = pl.kernel(
      _bin_body,
      out_type=(
          jax.ShapeDtypeStruct((NW, NCH, NBMAX, 2, BB), jnp.int32),
          jax.ShapeDtypeStruct((NW, 16), jnp.int32),
      ),
      mesh=_mesh(),
      scratch_types=[
          pltpu.VMEM((EW,), jnp.int32),
          pltpu.VMEM((EW,), jnp.int32),
          pltpu.VMEM((NBMAX, 2, BB), jnp.int32),
          pltpu.VMEM((16,), jnp.int32),
      ],
      compiler_params=pltpu.CompilerParams(needs_layout_passes=False),
  )
  return f(u_all, v_all)


def _combine_body(yctr_hbm, yed_hbm, buv_hbm, cnt_hbm, temp_hbm,
                  acc, ixb, rows, cnt_v, gsem, ssem):
  cid = lax.axis_index("c")
  sid = lax.axis_index("s")
  lane = lax.iota(jnp.int32, 16)
  rps = CH // NSUB
  for wj in range(NW // NSUB):
    pltpu.sync_copy(cnt_hbm.at[sid * (NW // NSUB) + wj], cnt_v.at[wj])
  for c in range(NCH):
    @pl.when((c % NC) == cid)
    def _():
      base = c * CH
      pltpu.sync_copy(
          yctr_hbm.at[pl.ds(base + sid * rps, rps)],
          acc.at[pl.ds(sid * rps, rps)])
      plsc.subcore_barrier()
      for wj in range(NW // NSUB):
        w = sid * (NW // NSUB) + wj
        kcnt = jnp.max(jnp.where(lane == c, cnt_v[wj], 0))
        nb = kcnt // BB

        def body(sb, carry):
          pltpu.sync_copy(buv_hbm.at[w, c, pl.ds(sb * 8, 8)], ixb)
          for h in range(2):
            for k in range(4):
              j = sb * 8 + h * 4 + k
              @pl.when(j < nb)
              def _():
                q = h * 4 + k
                pltpu.async_copy(
                    yed_hbm.at[ixb.at[q, 1]], rows.at[k], gsem.at[k])
            for k in range(4):
              j = sb * 8 + h * 4 + k
              @pl.when(j < nb)
              def _():
                q = h * 4 + k
                pltpu.make_async_copy(
                    yed_hbm.at[ixb.at[q, 1]], rows.at[k], gsem.at[k]).wait()
                pltpu.async_copy(
                    rows.at[k], acc.at[ixb.at[q, 0]],
                    ssem.at[k], add=True)
            for k in range(4):
              j = sb * 8 + h * 4 + k
              @pl.when(j < nb)
              def _():
                q = h * 4 + k
                pltpu.make_async_copy(
                    rows.at[k], acc.at[ixb.at[q, 0]],
                    ssem.at[k]).wait()
          return carry

        lax.fori_loop(0, (nb + 7) // 8, body, jnp.int32(0))
      plsc.subcore_barrier()
      pltpu.sync_copy(
          acc.at[pl.ds(sid * rps, rps)],
          temp_hbm.at[pl.ds(base + sid * rps, rps)])


def _combine(yctr, yed, buv, cnt):
  f = pl.kernel(
      _combine_body,
      out_type=jax.ShapeDtypeStruct((NPAD, D), jnp.float32),
      mesh=_mesh(),
      scratch_types=[
          pltpu.VMEM_SHARED((CH + 8, D), jnp.float32),
          pltpu.VMEM((8, 2, BB), jnp.int32),
          pltpu.VMEM((4, BB, D), jnp.float32),
          pltpu.VMEM((2, 16), jnp.int32),
          pltpu.SemaphoreType.DMA((4,)),
          pltpu.SemaphoreType.DMA((4,)),
      ],
      compiler_params=pltpu.CompilerParams(needs_layout_passes=False),
  )
  return f(yctr, yed, buv, cnt)


def _in_body(c_ref, f_ref, iw1, ib1, iw2, ib2, sw1, sb1, sw2, sb2, o_ref):
  c = c_ref[...]
  f = f_ref[...]
  h1 = jnp.maximum(c[:, 0:1] * iw1[0:1, :] + c[:, 1:2] * iw1[1:2, :]
                   + ib1[...], 0.0)
  y1 = jnp.dot(h1, iw2[...], preferred_element_type=jnp.float32) + ib2[...]
  h2 = jnp.maximum(f[:, 0:1] * sw1[0:1, :] + f[:, 1:2] * sw1[1:2, :]
                   + sb1[...], 0.0)
  y2 = jnp.dot(h2, sw2[...], preferred_element_type=jnp.float32) + sb2[...]
  o_ref[...] = jnp.maximum(y1 + y2, 0.0)


def _input_mlp(ctrs_p, feats_p, iw1, ib1, iw2, ib2, sw1, sb1, sw2, sb2):
  full = lambda s: pl.BlockSpec(s, lambda j: tuple(0 for _ in s))
  return pl.pallas_call(
      _in_body,
      grid=(NBLK,),
      in_specs=[
          pl.BlockSpec((BLK, 2), lambda j: (j, 0)),
          pl.BlockSpec((BLK, 2), lambda j: (j, 0)),
          full((2, D)), full((1, D)), full((D, D)), full((1, D)),
          full((2, D)), full((1, D)), full((D, D)), full((1, D)),
      ],
      out_specs=pl.BlockSpec((BLK, D), lambda j: (j, 0)),
      out_shape=jax.ShapeDtypeStruct((NPAD, D), jnp.float32),
  )(ctrs_p, feats_p, iw1, ib1, iw2, ib2, sw1, sb1, sw2, sb2)


def _yc_body(f_ref, w_ref, o_ref):
  o_ref[...] = jnp.dot(f_ref[...], w_ref[...], preferred_element_type=jnp.float32)


def _y_ctr(feat, w):
  full = lambda s: pl.BlockSpec(s, lambda j: tuple(0 for _ in s))
  return pl.pallas_call(
      _yc_body,
      grid=(NBLK,),
      in_specs=[pl.BlockSpec((BLK, D), lambda j: (j, 0)), full((D, D))],
      out_specs=pl.BlockSpec((BLK, D), lambda j: (j, 0)),
      out_shape=jax.ShapeDtypeStruct((NPAD, D), jnp.float32),
  )(feat, w)


def _ye_body(f_ref, w_ref, o_ref):
  o_ref[...] = jnp.dot(f_ref[...].astype(jnp.bfloat16),
                       w_ref[0].astype(jnp.bfloat16),
                       preferred_element_type=jnp.float32)


def _y_edges(feat, wed):
  return pl.pallas_call(
      _ye_body,
      grid=(NBLK, 14),
      in_specs=[
          pl.BlockSpec((BLK, D), lambda j, k: (j, 0)),
          pl.BlockSpec((1, D, D), lambda j, k: (k, 0, 0)),
      ],
      out_specs=pl.BlockSpec((BLK, D), lambda j, k: (k * NBLK + j, 0)),
      out_shape=jax.ShapeDtypeStruct((14 * NPAD, D), jnp.float32),
  )(feat, wed)


def _post_body(t_ref, r_ref, g_ref, b_ref, w1, b1, w2, b2, o_ref):
  t = t_ref[...]
  mu = jnp.mean(t, axis=1, keepdims=True)
  var = jnp.mean((t - mu) * (t - mu), axis=1, keepdims=True)
  xn = (t - mu) * lax.rsqrt(var + 1e-5) * g_ref[...] + b_ref[...]
  h = jnp.maximum(xn, 0.0)
  h = jnp.maximum(jnp.dot(h, w1[...], preferred_element_type=jnp.float32)
                  + b1[...], 0.0)
  y = jnp.dot(h, w2[...], preferred_element_type=jnp.float32) + b2[...]
  o_ref[...] = jnp.maximum(y + r_ref[...], 0.0)


def _post(temp, res, g, b, w1, b1, w2, b2):
  full = lambda s: pl.BlockSpec(s, lambda j: tuple(0 for _ in s))
  return pl.pallas_call(
      _post_body,
      grid=(NBLK,),
      in_specs=[
          pl.BlockSpec((BLK, D), lambda j: (j, 0)),
          pl.BlockSpec((BLK, D), lambda j: (j, 0)),
          full((1, D)), full((1, D)), full((D, D)), full((1, D)),
          full((D, D)), full((1, D)),
      ],
      out_specs=pl.BlockSpec((BLK, D), lambda j: (j, 0)),
      out_shape=jax.ShapeDtypeStruct((NPAD, D), jnp.float32),
  )(temp, res, g, b, w1, b1, w2, b2)


def kernel(feats, ctrs, pre_u, pre_v, suc_u, suc_v, left_u, left_v, right_u,
           right_v, idcs, in_W1, in_b1, in_W2, in_b2, seg_W1, seg_b1, seg_W2,
           seg_b2, ctr_W, pre_W, suc_W, left_W, right_W, gn_g, gn_b, ctr2_W1,
           ctr2_b1, ctr2_W2, ctr2_b2):
  i32 = jnp.int32
  pad = lambda x: jnp.pad(x, ((0, NPAD - N), (0, 0)))
  ctrs_p = pad(ctrs)
  feats_p = pad(feats)

  u_all = jnp.concatenate([
      pre_u.reshape(-1), suc_u.reshape(-1), left_u, right_u]).astype(i32)
  pre_off = jnp.arange(NS, dtype=i32)[:, None] * NPAD
  suc_off = (NS + jnp.arange(NS, dtype=i32))[:, None] * NPAD
  v_all = jnp.concatenate([
      (pre_v.astype(i32) + pre_off).reshape(-1),
      (suc_v.astype(i32) + suc_off).reshape(-1),
      left_v.astype(i32) + 2 * NS * NPAD,
      right_v.astype(i32) + (2 * NS + 1) * NPAD,
  ])
  u_all = jnp.pad(u_all, (0, E_PAD - E_TOT), constant_values=2**30)
  v_all = jnp.pad(v_all, (0, E_PAD - E_TOT))

  wed = jnp.concatenate([
      pre_W, suc_W, left_W[:, None], right_W[:, None]], axis=1)

  r2 = lambda x: x.reshape(1, -1)

  buv, cnt = _bin_kernel(u_all, v_all)

  feat = _input_mlp(ctrs_p, feats_p, in_W1, r2(in_b1), in_W2, r2(in_b2),
                    seg_W1, r2(seg_b1), seg_W2, r2(seg_b2))
  res = feat
  for i in range(4):
    yctr = _y_ctr(feat, ctr_W[i])
    yed = _y_edges(feat, wed[i])
    temp = _combine(yctr, yed, buv, cnt)
    feat = _post(temp, res, r2(gn_g[i]), r2(gn_b[i]), ctr2_W1[i],
                 r2(ctr2_b1[i]), ctr2_W2[i], r2(ctr2_b2[i]))
    res = feat
  return feat[:N]
